# baseline (speedup 1.0000x reference)
import functools

import jax
import jax.numpy as jnp
from jax import lax
from jax.experimental import pallas as pl
from jax.experimental.pallas import tpu as pltpu
from jax.experimental.pallas import tpu_sc as plsc

NUM_RELATIONS = 1000
DIM = 32
BATCH = 16384
NC = 2
NS = 16
NW = NC * NS
B_PER_W = BATCH // NW
IDX_CHUNK = 64
N_CHUNKS = B_PER_W // IDX_CHUNK
QSTRIDE = 17


@functools.partial(
    pl.kernel,
    out_type=jax.ShapeDtypeStruct((BATCH,), jnp.float32),
    mesh=plsc.VectorSubcoreMesh(core_axis_name="c", subcore_axis_name="s"),
    compiler_params=pltpu.CompilerParams(
        needs_layout_passes=False, use_tc_tiling_on_sc=False,
        skip_device_barrier=True, disable_bounds_checks=True,
        disable_semaphore_checks=True),
    scratch_types=[
        pltpu.VMEM((N_CHUNKS, IDX_CHUNK), jnp.int32),
        pltpu.VMEM((B_PER_W, DIM), jnp.float32),
        pltpu.VMEM((B_PER_W, DIM), jnp.float32),
        pltpu.VMEM((B_PER_W, DIM), jnp.float32),
        pltpu.VMEM((B_PER_W * QSTRIDE,), jnp.float32),
        pltpu.VMEM((B_PER_W,), jnp.float32),
    ] + [pltpu.SemaphoreType.DMA] * (N_CHUNKS + 1),
)
def _dist_mult(e_h_hbm, r_hbm, e_t_hbm, w_hbm, out_hbm,
               idx_v, h_v, w_v, t_v, q_v, out_v, *sems):
    wid = lax.axis_index("s") * NC + lax.axis_index("c")
    base = wid * B_PER_W
    out_sem = sems[N_CHUNKS]

    ht_copies = []
    for s in range(N_CHUNKS):
        off = s * IDX_CHUNK
        ht_copies.append([
            pltpu.async_copy(e_h_hbm.at[pl.ds(base + off, IDX_CHUNK)],
                             h_v.at[pl.ds(off, IDX_CHUNK)], sems[s]),
            pltpu.async_copy(e_t_hbm.at[pl.ds(base + off, IDX_CHUNK)],
                             t_v.at[pl.ds(off, IDX_CHUNK)], sems[s]),
        ])
    pltpu.sync_copy(r_hbm.at[pl.ds(wid * N_CHUNKS, N_CHUNKS)], idx_v)
    gathers = [
        pltpu.async_copy(w_hbm.at[idx_v.at[s]],
                         w_v.at[pl.ds(s * IDX_CHUNK, IDX_CHUNK)], sems[s])
        for s in range(N_CHUNKS)
    ]

    lanes = lax.iota(jnp.int32, 16)
    qcol = lanes * QSTRIDE

    def group(g):
        rbase = g * 16
        qoff = g * (16 * QSTRIDE)
        for i in range(16):
            row = rbase + i
            h0 = h_v[row, pl.ds(0, 16)]
            h1 = h_v[row, pl.ds(16, 16)]
            w0 = w_v[row, pl.ds(0, 16)]
            w1 = w_v[row, pl.ds(16, 16)]
            t0 = t_v[row, pl.ds(0, 16)]
            t1 = t_v[row, pl.ds(16, 16)]
            q_v[pl.ds(qoff + i * QSTRIDE, 16)] = h0 * w0 * t0 + h1 * w1 * t1
        cols = [plsc.load_gather(q_v, [qoff + qcol + d]) for d in range(16)]
        while len(cols) > 1:
            cols = [cols[k] + cols[k + 1] for k in range(0, len(cols), 2)]
        out_v[pl.ds(rbase, 16)] = cols[0]

    groups_per_stage = IDX_CHUNK // 16
    out_copies = []
    for s in range(N_CHUNKS):
        for cp in ht_copies[s]:
            cp.wait()
        gathers[s].wait()
        goff = s * groups_per_stage
        plsc.parallel_loop(goff, goff + groups_per_stage, unroll=2)(group)
        off = s * IDX_CHUNK
        out_copies.append(
            pltpu.async_copy(out_v.at[pl.ds(off, IDX_CHUNK)],
                             out_hbm.at[pl.ds(base + off, IDX_CHUNK)],
                             out_sem))
    for cp in out_copies:
        cp.wait()


def kernel(e_h, r, e_t, rel_weight):
    r2 = jnp.reshape(r.astype(jnp.int32), (BATCH // IDX_CHUNK, IDX_CHUNK))
    return _dist_mult(e_h, r2, e_t, rel_weight)


# speedup vs baseline: 1.0312x; 1.0312x over previous
import functools

import jax
import jax.numpy as jnp
from jax import lax
from jax.experimental import pallas as pl
from jax.experimental.pallas import tpu as pltpu
from jax.experimental.pallas import tpu_sc as plsc

NUM_RELATIONS = 1000
DIM = 32
BATCH = 16384
NC = 2
NS = 16
NW = NC * NS
B_PER_W = BATCH // NW
IDX_CHUNK = 128
N_CHUNKS = B_PER_W // IDX_CHUNK
QSTRIDE = 17


@functools.partial(
    pl.kernel,
    out_type=jax.ShapeDtypeStruct((BATCH,), jnp.float32),
    mesh=plsc.VectorSubcoreMesh(core_axis_name="c", subcore_axis_name="s"),
    compiler_params=pltpu.CompilerParams(
        needs_layout_passes=False, use_tc_tiling_on_sc=False,
        skip_device_barrier=True, disable_bounds_checks=True,
        disable_semaphore_checks=True),
    scratch_types=[
        pltpu.VMEM((N_CHUNKS, IDX_CHUNK), jnp.int32),
        pltpu.VMEM((B_PER_W, DIM), jnp.float32),
        pltpu.VMEM((B_PER_W, DIM), jnp.float32),
        pltpu.VMEM((B_PER_W, DIM), jnp.float32),
        pltpu.VMEM((B_PER_W * QSTRIDE,), jnp.float32),
        pltpu.VMEM((B_PER_W,), jnp.float32),
    ] + [pltpu.SemaphoreType.DMA] * (N_CHUNKS + 1),
)
def _dist_mult(e_h_hbm, r_hbm, e_t_hbm, w_hbm, out_hbm,
               idx_v, h_v, w_v, t_v, q_v, out_v, *sems):
    wid = lax.axis_index("s") * NC + lax.axis_index("c")
    base = wid * B_PER_W
    out_sem = sems[N_CHUNKS]

    ht_copies = []
    for s in range(N_CHUNKS):
        off = s * IDX_CHUNK
        ht_copies.append([
            pltpu.async_copy(e_h_hbm.at[pl.ds(base + off, IDX_CHUNK)],
                             h_v.at[pl.ds(off, IDX_CHUNK)], sems[s]),
            pltpu.async_copy(e_t_hbm.at[pl.ds(base + off, IDX_CHUNK)],
                             t_v.at[pl.ds(off, IDX_CHUNK)], sems[s]),
        ])
    pltpu.sync_copy(r_hbm.at[pl.ds(wid * N_CHUNKS, N_CHUNKS)], idx_v)
    gathers = [
        pltpu.async_copy(w_hbm.at[idx_v.at[s]],
                         w_v.at[pl.ds(s * IDX_CHUNK, IDX_CHUNK)], sems[s])
        for s in range(N_CHUNKS)
    ]

    lanes = lax.iota(jnp.int32, 16)
    qcol = lanes * QSTRIDE

    def group(g):
        rbase = g * 16
        qoff = g * (16 * QSTRIDE)
        for i in range(16):
            row = rbase + i
            h0 = h_v[row, pl.ds(0, 16)]
            h1 = h_v[row, pl.ds(16, 16)]
            w0 = w_v[row, pl.ds(0, 16)]
            w1 = w_v[row, pl.ds(16, 16)]
            t0 = t_v[row, pl.ds(0, 16)]
            t1 = t_v[row, pl.ds(16, 16)]
            q_v[pl.ds(qoff + i * QSTRIDE, 16)] = h0 * w0 * t0 + h1 * w1 * t1
        cols = [plsc.load_gather(q_v, [qoff + qcol + d]) for d in range(16)]
        while len(cols) > 1:
            cols = [cols[k] + cols[k + 1] for k in range(0, len(cols), 2)]
        out_v[pl.ds(rbase, 16)] = cols[0]

    groups_per_stage = IDX_CHUNK // 16
    out_copies = []
    for s in range(N_CHUNKS):
        for cp in ht_copies[s]:
            cp.wait()
        gathers[s].wait()
        goff = s * groups_per_stage
        plsc.parallel_loop(goff, goff + groups_per_stage, unroll=2)(group)
        off = s * IDX_CHUNK
        out_copies.append(
            pltpu.async_copy(out_v.at[pl.ds(off, IDX_CHUNK)],
                             out_hbm.at[pl.ds(base + off, IDX_CHUNK)],
                             out_sem))
    for cp in out_copies:
        cp.wait()


def kernel(e_h, r, e_t, rel_weight):
    r2 = jnp.reshape(r.astype(jnp.int32), (BATCH // IDX_CHUNK, IDX_CHUNK))
    return _dist_mult(e_h, r2, e_t, rel_weight)


# speedup vs baseline: 1.0415x; 1.0100x over previous
import functools

import jax
import jax.numpy as jnp
from jax import lax
from jax.experimental import pallas as pl
from jax.experimental.pallas import tpu as pltpu
from jax.experimental.pallas import tpu_sc as plsc

NUM_RELATIONS = 1000
DIM = 32
BATCH = 16384
NC = 2
NS = 16
NW = NC * NS
B_PER_W = BATCH // NW
IDX_CHUNK = 128
N_CHUNKS = B_PER_W // IDX_CHUNK
QSTRIDE = 17


@functools.partial(
    pl.kernel,
    out_type=jax.ShapeDtypeStruct((BATCH,), jnp.float32),
    mesh=plsc.VectorSubcoreMesh(core_axis_name="c", subcore_axis_name="s"),
    compiler_params=pltpu.CompilerParams(
        needs_layout_passes=False, use_tc_tiling_on_sc=False,
        skip_device_barrier=True, disable_bounds_checks=True,
        disable_semaphore_checks=True),
    scratch_types=[
        pltpu.VMEM((N_CHUNKS, IDX_CHUNK), jnp.int32),
        pltpu.VMEM((B_PER_W, DIM), jnp.float32),
        pltpu.VMEM((B_PER_W, DIM), jnp.float32),
        pltpu.VMEM((B_PER_W, DIM), jnp.float32),
        pltpu.VMEM((B_PER_W * QSTRIDE,), jnp.float32),
        pltpu.VMEM((B_PER_W,), jnp.float32),
        pltpu.SemaphoreType.DMA,
        pltpu.SemaphoreType.DMA,
        pltpu.SemaphoreType.DMA,
        pltpu.SemaphoreType.DMA,
        pltpu.SemaphoreType.DMA,
    ],
)
def _dist_mult(e_h_hbm, r_hbm, e_t_hbm, w_hbm, out_hbm,
               idx_v, h_v, w_v, t_v, q_v, out_v, *sems):
    wid = lax.axis_index("s") * NC + lax.axis_index("c")
    base = wid * B_PER_W

    pltpu.sync_copy(r_hbm.at[pl.ds(wid * N_CHUNKS, N_CHUNKS)], idx_v)

    def fire(s):
        off = s * IDX_CHUNK
        return [
            pltpu.async_copy(w_hbm.at[idx_v.at[s]],
                             w_v.at[pl.ds(off, IDX_CHUNK)], sems[s]),
            pltpu.async_copy(e_h_hbm.at[pl.ds(base + off, IDX_CHUNK)],
                             h_v.at[pl.ds(off, IDX_CHUNK)], sems[s]),
            pltpu.async_copy(e_t_hbm.at[pl.ds(base + off, IDX_CHUNK)],
                             t_v.at[pl.ds(off, IDX_CHUNK)], sems[s]),
        ]

    lanes = lax.iota(jnp.int32, 16)
    qcol = lanes * QSTRIDE

    def group(g):
        rbase = g * 16
        qoff = g * (16 * QSTRIDE)
        for i in range(16):
            row = rbase + i
            h0 = h_v[row, pl.ds(0, 16)]
            h1 = h_v[row, pl.ds(16, 16)]
            w0 = w_v[row, pl.ds(0, 16)]
            w1 = w_v[row, pl.ds(16, 16)]
            t0 = t_v[row, pl.ds(0, 16)]
            t1 = t_v[row, pl.ds(16, 16)]
            q_v[pl.ds(qoff + i * QSTRIDE, 16)] = h0 * w0 * t0 + h1 * w1 * t1
        cols = [plsc.load_gather(q_v, [qoff + qcol + d]) for d in range(16)]
        while len(cols) > 1:
            cols = [cols[k] + cols[k + 1] for k in range(0, len(cols), 2)]
        out_v[pl.ds(rbase, 16)] = cols[0]

    groups_per_stage = IDX_CHUNK // 16
    out_sem = sems[N_CHUNKS]
    pending = fire(0)
    out_copies = []
    for s in range(N_CHUNKS):
        nxt = fire(s + 1) if s + 1 < N_CHUNKS else []
        for cp in pending:
            cp.wait()
        pending = nxt
        goff = s * groups_per_stage
        plsc.parallel_loop(goff, goff + groups_per_stage, unroll=2)(group)
        off = s * IDX_CHUNK
        out_copies.append(
            pltpu.async_copy(out_v.at[pl.ds(off, IDX_CHUNK)],
                             out_hbm.at[pl.ds(base + off, IDX_CHUNK)],
                             out_sem))
    for cp in out_copies:
        cp.wait()


def kernel(e_h, r, e_t, rel_weight):
    r2 = jnp.reshape(r.astype(jnp.int32), (BATCH // IDX_CHUNK, IDX_CHUNK))
    return _dist_mult(e_h, r2, e_t, rel_weight)


# speedup vs baseline: 1.3172x; 1.2647x over previous
import functools

import jax
import jax.numpy as jnp
from jax import lax
from jax.experimental import pallas as pl
from jax.experimental.pallas import tpu as pltpu
from jax.experimental.pallas import tpu_sc as plsc

NUM_RELATIONS = 1000
DIM = 32
BATCH = 16384
NC = 2
NS = 16
NW = NC * NS
B_PER_W = BATCH // NW
IDX_CHUNK = 128
N_CHUNKS = B_PER_W // IDX_CHUNK
QSTRIDE = 17


@functools.partial(
    pl.kernel,
    out_type=jax.ShapeDtypeStruct((BATCH,), jnp.float32),
    mesh=plsc.VectorSubcoreMesh(core_axis_name="c", subcore_axis_name="s"),
    compiler_params=pltpu.CompilerParams(
        needs_layout_passes=False, use_tc_tiling_on_sc=False,
        skip_device_barrier=True, disable_bounds_checks=True,
        disable_semaphore_checks=True),
    scratch_types=[
        pltpu.VMEM((N_CHUNKS, IDX_CHUNK), jnp.int32),
        pltpu.VMEM((B_PER_W * DIM,), jnp.float32),
        pltpu.VMEM((B_PER_W, DIM), jnp.float32),
        pltpu.VMEM((B_PER_W * QSTRIDE,), jnp.float32),
        pltpu.VMEM((B_PER_W,), jnp.float32),
        pltpu.SemaphoreType.DMA,
        pltpu.SemaphoreType.DMA,
        pltpu.SemaphoreType.DMA,
        pltpu.SemaphoreType.DMA,
        pltpu.SemaphoreType.DMA,
    ],
)
def _dist_mult(u_hbm, r_hbm, w_hbm, out_hbm,
               idx_v, u_v, w_v, q_v, out_v, *sems):
    wid = lax.axis_index("s") * NC + lax.axis_index("c")
    base = wid * B_PER_W

    pltpu.sync_copy(r_hbm.at[pl.ds(wid * N_CHUNKS, N_CHUNKS)], idx_v)

    def fire(s):
        off = s * IDX_CHUNK
        return [
            pltpu.async_copy(w_hbm.at[idx_v.at[s]],
                             w_v.at[pl.ds(off, IDX_CHUNK)], sems[s]),
            pltpu.async_copy(u_hbm.at[pl.ds((base + off) * DIM,
                                            IDX_CHUNK * DIM)],
                             u_v.at[pl.ds(off * DIM, IDX_CHUNK * DIM)],
                             sems[s]),
        ]

    lanes = lax.iota(jnp.int32, 16)
    qcol = lanes * QSTRIDE

    def group(g):
        rbase = g * 16
        qoff = g * (16 * QSTRIDE)
        for i in range(16):
            row = rbase + i
            u0 = u_v[pl.ds(row * DIM, 16)]
            u1 = u_v[pl.ds(row * DIM + 16, 16)]
            w0 = w_v[row, pl.ds(0, 16)]
            w1 = w_v[row, pl.ds(16, 16)]
            q_v[pl.ds(qoff + i * QSTRIDE, 16)] = u0 * w0 + u1 * w1
        cols = [plsc.load_gather(q_v, [qoff + qcol + d]) for d in range(16)]
        while len(cols) > 1:
            cols = [cols[k] + cols[k + 1] for k in range(0, len(cols), 2)]
        out_v[pl.ds(rbase, 16)] = cols[0]

    groups_per_stage = IDX_CHUNK // 16
    out_sem = sems[N_CHUNKS]
    pending = fire(0)
    out_copies = []
    for s in range(N_CHUNKS):
        nxt = fire(s + 1) if s + 1 < N_CHUNKS else []
        for cp in pending:
            cp.wait()
        pending = nxt
        goff = s * groups_per_stage
        plsc.parallel_loop(goff, goff + groups_per_stage, unroll=2)(group)
        off = s * IDX_CHUNK
        out_copies.append(
            pltpu.async_copy(out_v.at[pl.ds(off, IDX_CHUNK)],
                             out_hbm.at[pl.ds(base + off, IDX_CHUNK)],
                             out_sem))
    for cp in out_copies:
        cp.wait()


def kernel(e_h, r, e_t, rel_weight):
    u = jnp.reshape(e_h * e_t, (BATCH * DIM,))
    r2 = jnp.reshape(r.astype(jnp.int32), (BATCH // IDX_CHUNK, IDX_CHUNK))
    return _dist_mult(u, r2, rel_weight)


# speedup vs baseline: 1.3210x; 1.0029x over previous
import functools

import jax
import jax.numpy as jnp
from jax import lax
from jax.experimental import pallas as pl
from jax.experimental.pallas import tpu as pltpu
from jax.experimental.pallas import tpu_sc as plsc

NUM_RELATIONS = 1000
DIM = 32
BATCH = 16384
NC = 2
NS = 16
NW = NC * NS
B_PER_W = BATCH // NW
IDX_CHUNK = 128
N_CHUNKS = B_PER_W // IDX_CHUNK
QSTRIDE = 17


@functools.partial(
    pl.kernel,
    out_type=jax.ShapeDtypeStruct((BATCH,), jnp.float32),
    mesh=plsc.VectorSubcoreMesh(core_axis_name="c", subcore_axis_name="s"),
    compiler_params=pltpu.CompilerParams(
        needs_layout_passes=False, use_tc_tiling_on_sc=False,
        skip_device_barrier=True, disable_bounds_checks=True,
        disable_semaphore_checks=True),
    scratch_types=[
        pltpu.VMEM((N_CHUNKS, IDX_CHUNK), jnp.int32),
        pltpu.VMEM((B_PER_W * DIM // 128, 128), jnp.float32),
        pltpu.VMEM((B_PER_W, DIM), jnp.float32),
        pltpu.VMEM((B_PER_W * QSTRIDE,), jnp.float32),
        pltpu.VMEM((B_PER_W,), jnp.float32),
        pltpu.SemaphoreType.DMA,
        pltpu.SemaphoreType.DMA,
        pltpu.SemaphoreType.DMA,
        pltpu.SemaphoreType.DMA,
        pltpu.SemaphoreType.DMA,
    ],
)
def _dist_mult(u_hbm, r_hbm, w_hbm, out_hbm,
               idx_v, u_v, w_v, q_v, out_v, *sems):
    wid = lax.axis_index("s") * NC + lax.axis_index("c")
    base = wid * B_PER_W

    pltpu.sync_copy(r_hbm.at[pl.ds(wid * N_CHUNKS, N_CHUNKS)], idx_v)

    def fire(s):
        off = s * IDX_CHUNK
        return [
            pltpu.async_copy(w_hbm.at[idx_v.at[s]],
                             w_v.at[pl.ds(off, IDX_CHUNK)], sems[s]),
            pltpu.async_copy(u_hbm.at[pl.ds((base + off) * DIM // 128,
                                            IDX_CHUNK * DIM // 128)],
                             u_v.at[pl.ds(off * DIM // 128,
                                          IDX_CHUNK * DIM // 128)],
                             sems[s]),
        ]

    lanes = lax.iota(jnp.int32, 16)
    qcol = lanes * QSTRIDE

    def group(g):
        rbase = g * 16
        qoff = g * (16 * QSTRIDE)
        for i in range(16):
            row = rbase + i
            u0 = u_v[g * 4 + i // 4, pl.ds((i % 4) * DIM, 16)]
            u1 = u_v[g * 4 + i // 4, pl.ds((i % 4) * DIM + 16, 16)]
            w0 = w_v[row, pl.ds(0, 16)]
            w1 = w_v[row, pl.ds(16, 16)]
            q_v[pl.ds(qoff + i * QSTRIDE, 16)] = u0 * w0 + u1 * w1
        cols = [plsc.load_gather(q_v, [qoff + qcol + d]) for d in range(16)]
        while len(cols) > 1:
            cols = [cols[k] + cols[k + 1] for k in range(0, len(cols), 2)]
        out_v[pl.ds(rbase, 16)] = cols[0]

    groups_per_stage = IDX_CHUNK // 16
    out_sem = sems[N_CHUNKS]
    pending = fire(0)
    out_copies = []
    for s in range(N_CHUNKS):
        nxt = fire(s + 1) if s + 1 < N_CHUNKS else []
        for cp in pending:
            cp.wait()
        pending = nxt
        goff = s * groups_per_stage
        plsc.parallel_loop(goff, goff + groups_per_stage, unroll=2)(group)
        off = s * IDX_CHUNK
        out_copies.append(
            pltpu.async_copy(out_v.at[pl.ds(off, IDX_CHUNK)],
                             out_hbm.at[pl.ds(base + off, IDX_CHUNK)],
                             out_sem))
    for cp in out_copies:
        cp.wait()


def kernel(e_h, r, e_t, rel_weight):
    u = jnp.reshape(e_h * e_t, (BATCH * DIM // 128, 128))
    r2 = jnp.reshape(r.astype(jnp.int32), (BATCH // IDX_CHUNK, IDX_CHUNK))
    return _dist_mult(u, r2, rel_weight)
